# baseline (speedup 1.0000x reference)
import functools

import numpy as np
import jax
import jax.numpy as jnp
from jax import lax
from jax.experimental import pallas as pl
from jax.experimental.pallas import tpu as pltpu
from jax.experimental.pallas import tpu_sc as plsc

SEQ_LEN = 200
OUT_DIM = 64
SCALE = 8.0
CHUNK_A = 104
CHUNK_B = 96


def _pos_encoding(length, output_dim):
    depth = output_dim / 2
    positions = np.arange(length)[:, np.newaxis]
    depths = np.arange(depth)[np.newaxis, :] / depth
    angle_rates = 1 / 10000 ** depths
    angle_rads = positions * angle_rates
    return np.concatenate(
        [np.sin(angle_rads), np.cos(angle_rads)], axis=-1
    ).astype(np.float32)


_PE_CONST = jnp.asarray(_pos_encoding(SEQ_LEN, OUT_DIM))


def kernel(x, table):
    B, S = x.shape
    V, D = table.shape
    info = plsc.get_sparse_core_info()
    NC, NS = info.num_cores, info.num_subcores
    NW = NC * NS
    RPW = B // NW

    @functools.partial(
        pl.kernel,
        mesh=plsc.VectorSubcoreMesh(core_axis_name="c", subcore_axis_name="s"),
        compiler_params=pltpu.CompilerParams(use_tc_tiling_on_sc=False),
        out_type=jax.ShapeDtypeStruct((B * S, D), jnp.float32),
        scratch_types=[
            pltpu.VMEM((CHUNK_A,), jnp.int32),
            pltpu.VMEM((CHUNK_B,), jnp.int32),
            pltpu.VMEM((S, D), jnp.float32),
            pltpu.VMEM((S, D), jnp.float32),
            pltpu.SemaphoreType.DMA,
        ],
    )
    def run(table_hbm, x_hbm, pe_hbm, out_hbm, idx_a, idx_b, rows_v, pe_v, sem):
        wid = lax.axis_index("s") * NC + lax.axis_index("c")
        base = wid * RPW
        pltpu.sync_copy(pe_hbm, pe_v)

        def row_body(i, carry):
            r = base + i
            pltpu.sync_copy(x_hbm.at[pl.ds(r * S, CHUNK_A)], idx_a)
            pltpu.sync_copy(x_hbm.at[pl.ds(r * S + CHUNK_A, CHUNK_B)], idx_b)
            pltpu.async_copy(
                table_hbm.at[idx_a], rows_v.at[pl.ds(0, CHUNK_A)], sem
            ).wait()
            pltpu.async_copy(
                table_hbm.at[idx_b], rows_v.at[pl.ds(CHUNK_A, CHUNK_B)], sem
            ).wait()

            def seq_body(s, c2):
                for d in range(D // 16):
                    sl = pl.ds(d * 16, 16)
                    rows_v[s, sl] = rows_v[s, sl] * SCALE + pe_v[s, sl]
                return c2

            lax.fori_loop(0, S, seq_body, 0)
            pltpu.sync_copy(rows_v, out_hbm.at[pl.ds(r * S, S)])
            return carry

        lax.fori_loop(0, RPW, row_body, 0)

    out = run(table, x.reshape(-1), _PE_CONST)
    return out.reshape(B, S, D)


# speedup vs baseline: 1.3125x; 1.3125x over previous
import functools

import numpy as np
import jax
import jax.numpy as jnp
from jax import lax
from jax.experimental import pallas as pl
from jax.experimental.pallas import tpu as pltpu
from jax.experimental.pallas import tpu_sc as plsc

SEQ_LEN = 200
OUT_DIM = 64
SCALE = 8.0
CHUNK_A = 104
CHUNK_B = 96
NBUF = 4


def _pos_encoding(length, output_dim):
    depth = output_dim / 2
    positions = np.arange(length)[:, np.newaxis]
    depths = np.arange(depth)[np.newaxis, :] / depth
    angle_rates = 1 / 10000 ** depths
    angle_rads = positions * angle_rates
    return np.concatenate(
        [np.sin(angle_rads), np.cos(angle_rads)], axis=-1
    ).astype(np.float32)


_PE_CONST = jnp.asarray(_pos_encoding(SEQ_LEN, OUT_DIM))


def kernel(x, table):
    B, S = x.shape
    V, D = table.shape
    info = plsc.get_sparse_core_info()
    NC, NS = info.num_cores, info.num_subcores
    NW = NC * NS
    RPW = B // NW

    @functools.partial(
        pl.kernel,
        mesh=plsc.VectorSubcoreMesh(core_axis_name="c", subcore_axis_name="s"),
        compiler_params=pltpu.CompilerParams(use_tc_tiling_on_sc=False),
        out_type=jax.ShapeDtypeStruct((B * S, D), jnp.float32),
        scratch_types=[
            pltpu.VMEM((RPW * S,), jnp.int32),
            pltpu.VMEM((S, D), jnp.float32),
        ]
        + [pltpu.VMEM((S, D), jnp.float32) for _ in range(NBUF)]
        + [pltpu.SemaphoreType.DMA for _ in range(2 * NBUF)],
    )
    def run(table_hbm, x_hbm, pe_hbm, out_hbm, idx_all, pe_v, *bufs_and_sems):
        rows = bufs_and_sems[:NBUF]
        gsem = bufs_and_sems[NBUF : 2 * NBUF]
        wsem = bufs_and_sems[2 * NBUF : 3 * NBUF]

        wid = lax.axis_index("s") * NC + lax.axis_index("c")
        base = wid * RPW
        pltpu.sync_copy(x_hbm.at[pl.ds(base * S, RPW * S)], idx_all)
        pltpu.sync_copy(pe_hbm, pe_v)

        def gather_descs(g, b):
            o = g * S
            return (
                pltpu.make_async_copy(
                    table_hbm.at[idx_all.at[pl.ds(o, CHUNK_A)]],
                    rows[b].at[pl.ds(0, CHUNK_A)],
                    gsem[b],
                ),
                pltpu.make_async_copy(
                    table_hbm.at[idx_all.at[pl.ds(o + CHUNK_A, CHUNK_B)]],
                    rows[b].at[pl.ds(CHUNK_A, CHUNK_B)],
                    gsem[b],
                ),
            )

        def write_desc(g, b):
            return pltpu.make_async_copy(
                rows[b], out_hbm.at[pl.ds((base + g) * S, S)], wsem[b]
            )

        for d in gather_descs(0, 0):
            d.start()

        @pl.loop(0, RPW // NBUF)
        def _(j):
            for b in range(NBUF):
                g = j * NBUF + b
                nb = (b + 1) % NBUF

                @pl.when(g >= NBUF - 1)
                def _():
                    write_desc(g - (NBUF - 1), nb).wait()

                @pl.when(g + 1 < RPW)
                def _():
                    for d in gather_descs(g + 1, nb):
                        d.start()

                for d in gather_descs(g, b):
                    d.wait()

                @plsc.parallel_loop(0, S, unroll=8)
                def _(s):
                    for k in range(D // 16):
                        sl = pl.ds(k * 16, 16)
                        rows[b][s, sl] = rows[b][s, sl] * SCALE + pe_v[s, sl]

                write_desc(g, b).start()

        for g in range(RPW - NBUF + 1, RPW):
            write_desc(g, g % NBUF).wait()

    out = run(table, x.reshape(-1), _PE_CONST)
    return out.reshape(B, S, D)
